# baseline (speedup 1.0000x reference)
import jax
import jax.numpy as jnp
from jax import lax
from jax.experimental import pallas as pl
from jax.experimental.pallas import tpu as pltpu
from jax.experimental.pallas import tpu_sc as plsc

B, H, W, C = 8, 224, 224, 96
HO, WO = 224, 224
NC, NS = 2, 16
NW = NC * NS
PPW = (B * HO * WO) // NW
CHUNK = 128
CHUNKS_PER_W = PPW // CHUNK
NVEC = CHUNK // 16
C2 = 2 * C


def _body(tab_hbm, x_hbm, y_hbm, out_hbm,
          xv, yv, idx0, idx1, w_v, gbuf, obuf, sem):
  wid = lax.axis_index("s") * NC + lax.axis_index("c")
  p0 = wid * PPW
  pltpu.sync_copy(x_hbm.at[pl.ds(p0, PPW)], xv)
  pltpu.sync_copy(y_hbm.at[pl.ds(p0, PPW)], yv)
  base = (wid // 4) * (H * W)

  def stage_gathers(k, slot):
    def vec_body(v, carry2):
      s = pl.ds(v * 16, 16)
      sl = pl.ds(k * CHUNK + v * 16, 16)
      x = xv[sl]
      y = yv[sl]
      px = (0.5 * (x + 1.0)) * jnp.float32(W)
      py = (0.5 * (y + 1.0)) * jnp.float32(H)
      xi = px.astype(jnp.int32)
      yi = py.astype(jnp.int32)
      xm = jnp.clip(xi, 0, W - 1)
      xM = jnp.clip(xi + 1, 0, W - 1)
      ym = jnp.clip(yi, 0, H - 1)
      yM = jnp.clip(yi + 1, 0, H - 1)
      okf = jnp.where(xM > xm, jnp.float32(1.0), jnp.float32(0.0))
      wx0 = (xM.astype(jnp.float32) - px) * okf
      wx1 = (px - xm.astype(jnp.float32)) * okf
      wy0 = yM.astype(jnp.float32) - py
      wy1 = py - ym.astype(jnp.float32)
      idx0[slot, s] = base + ym * W + xm
      idx1[slot, s] = base + yM * W + xm
      w_v[slot, 0, s] = wx0 * wy0
      w_v[slot, 1, s] = wx0 * wy1
      w_v[slot, 2, s] = wx1 * wy0
      w_v[slot, 3, s] = wx1 * wy1
      return carry2

    lax.fori_loop(0, NVEC, vec_body, 0)
    pltpu.async_copy(tab_hbm.at[idx0.at[slot]], gbuf.at[slot, 0], sem.at[slot])
    pltpu.async_copy(tab_hbm.at[idx1.at[slot]], gbuf.at[slot, 1], sem.at[slot])

  def wait_gathers(slot):
    pltpu.make_async_copy(tab_hbm.at[idx0.at[slot]], gbuf.at[slot, 0],
                          sem.at[slot]).wait()
    pltpu.make_async_copy(tab_hbm.at[idx1.at[slot]], gbuf.at[slot, 1],
                          sem.at[slot]).wait()

  stage_gathers(0, 0)

  def chunk_body(k, carry):
    slot = k % 2
    nslot = 1 - slot

    @pl.when(k + 1 < CHUNKS_PER_W)
    def _():
      stage_gathers(k + 1, nslot)

    wait_gathers(slot)

    lane2 = lax.iota(jnp.int32, 16) * 2

    def px_body(i, carry2):
      wA = w_v[slot, 0, pl.ds(i, 16)][0]
      wB = w_v[slot, 1, pl.ds(i, 16)][0]
      wC = w_v[slot, 2, pl.ds(i, 16)][0]
      wD = w_v[slot, 3, pl.ds(i, 16)][0]
      orow = obuf.at[i]
      for c in range(C // 32):
        sA = pl.ds(c * 32, 32)
        sC = pl.ds(C + c * 32, 32)
        aE, aO = plsc.unpack(gbuf[slot, 0, i, sA],
                             format=plsc.PackFormat.INTERLEAVED)
        cE, cO = plsc.unpack(gbuf[slot, 0, i, sC],
                             format=plsc.PackFormat.INTERLEAVED)
        bE, bO = plsc.unpack(gbuf[slot, 1, i, sA],
                             format=plsc.PackFormat.INTERLEAVED)
        dE, dO = plsc.unpack(gbuf[slot, 1, i, sC],
                             format=plsc.PackFormat.INTERLEAVED)
        plsc.store_scatter(orow, [lane2 + (c * 32)],
                           aE * wA + bE * wB + cE * wC + dE * wD)
        plsc.store_scatter(orow, [lane2 + (c * 32 + 1)],
                           aO * wA + bO * wB + cO * wC + dO * wD)
      return carry2

    lax.fori_loop(0, CHUNK, px_body, 0)

    pltpu.sync_copy(obuf, out_hbm.at[pl.ds(p0 + k * CHUNK, CHUNK)])
    return carry

  lax.fori_loop(0, CHUNKS_PER_W, chunk_body, 0)


@jax.jit
def kernel(image, affine_transforms):
  flat = image.reshape(B * H * W, C).astype(jnp.bfloat16)
  flat_pad = jnp.concatenate([flat, flat[-1:]], axis=0)
  table2 = jnp.concatenate([flat_pad[:-1], flat_pad[1:]], axis=1)

  x_lin = jnp.linspace(-1.0, 1.0, WO)
  y_lin = jnp.linspace(-1.0, 1.0, HO)
  xg, yg = jnp.meshgrid(x_lin, y_lin)
  grid = jnp.concatenate(
      [xg.reshape(-1), yg.reshape(-1), jnp.ones(HO * WO, dtype=jnp.float32)],
      axis=0)
  grids = jnp.tile(grid.reshape(1, 3, HO * WO), (B, 1, 1))
  theta = affine_transforms.reshape(B, 2, 3)
  grids = jnp.einsum('bij,bjk->bik', theta, grids)
  x = grids[:, 0:1, :].reshape(-1).astype(jnp.float32)
  y = grids[:, 1:2, :].reshape(-1).astype(jnp.float32)

  call = pl.kernel(
      _body,
      out_type=jax.ShapeDtypeStruct((B * H * W, C), jnp.float32),
      mesh=plsc.VectorSubcoreMesh(core_axis_name="c", subcore_axis_name="s",
                                  num_cores=NC, num_subcores=NS),
      compiler_params=pltpu.CompilerParams(use_tc_tiling_on_sc=False,
                                           needs_layout_passes=False),
      scratch_types=[
          pltpu.VMEM((PPW,), jnp.float32),
          pltpu.VMEM((PPW,), jnp.float32),
          pltpu.VMEM((2, CHUNK), jnp.int32),
          pltpu.VMEM((2, CHUNK), jnp.int32),
          pltpu.VMEM((2, 4, CHUNK + 16), jnp.float32),
          pltpu.VMEM((2, 2, CHUNK, C2), jnp.bfloat16),
          pltpu.VMEM((CHUNK, C), jnp.float32),
          pltpu.SemaphoreType.DMA((2,)),
      ],
  )
  out = call(table2, x, y)
  return out.reshape(B, HO, WO, C)


# speedup vs baseline: 1.0009x; 1.0009x over previous
import jax
import jax.numpy as jnp
from jax import lax
from jax.experimental import pallas as pl
from jax.experimental.pallas import tpu as pltpu
from jax.experimental.pallas import tpu_sc as plsc

B, H, W, C = 8, 224, 224, 96
HO, WO = 224, 224
NC, NS = 2, 16
NW = NC * NS
PPW = (B * HO * WO) // NW
CHUNK = 112
CHUNKS_PER_W = PPW // CHUNK
NVEC = CHUNK // 16
C2 = 2 * C


def _body(tab_hbm, x_hbm, y_hbm, out_hbm,
          xv, yv, idx0, idx1, w_v, gbuf, obuf, sem):
  wid = lax.axis_index("s") * NC + lax.axis_index("c")
  p0 = wid * PPW
  pltpu.sync_copy(x_hbm.at[pl.ds(p0, PPW)], xv)
  pltpu.sync_copy(y_hbm.at[pl.ds(p0, PPW)], yv)
  base = (wid // 4) * (H * W)

  def stage_gathers(k, slot):
    def vec_body(v, carry2):
      s = pl.ds(v * 16, 16)
      sl = pl.ds(k * CHUNK + v * 16, 16)
      x = xv[sl]
      y = yv[sl]
      px = (0.5 * (x + 1.0)) * jnp.float32(W)
      py = (0.5 * (y + 1.0)) * jnp.float32(H)
      xi = px.astype(jnp.int32)
      yi = py.astype(jnp.int32)
      xm = jnp.clip(xi, 0, W - 1)
      xM = jnp.clip(xi + 1, 0, W - 1)
      ym = jnp.clip(yi, 0, H - 1)
      yM = jnp.clip(yi + 1, 0, H - 1)
      okf = jnp.where(xM > xm, jnp.float32(1.0), jnp.float32(0.0))
      wx0 = (xM.astype(jnp.float32) - px) * okf
      wx1 = (px - xm.astype(jnp.float32)) * okf
      wy0 = yM.astype(jnp.float32) - py
      wy1 = py - ym.astype(jnp.float32)
      idx0[slot, s] = base + ym * W + xm
      idx1[slot, s] = base + yM * W + xm
      w_v[slot, 0, s] = wx0 * wy0
      w_v[slot, 1, s] = wx0 * wy1
      w_v[slot, 2, s] = wx1 * wy0
      w_v[slot, 3, s] = wx1 * wy1
      return carry2

    lax.fori_loop(0, NVEC, vec_body, 0)
    pltpu.async_copy(tab_hbm.at[idx0.at[slot]], gbuf.at[slot, 0], sem.at[slot])
    pltpu.async_copy(tab_hbm.at[idx1.at[slot]], gbuf.at[slot, 1], sem.at[slot])

  def wait_gathers(slot):
    pltpu.make_async_copy(tab_hbm.at[idx0.at[slot]], gbuf.at[slot, 0],
                          sem.at[slot]).wait()
    pltpu.make_async_copy(tab_hbm.at[idx1.at[slot]], gbuf.at[slot, 1],
                          sem.at[slot]).wait()

  stage_gathers(0, 0)

  def chunk_body(k, carry):
    slot = k % 2
    nslot = 1 - slot

    @pl.when(k + 1 < CHUNKS_PER_W)
    def _():
      stage_gathers(k + 1, nslot)

    wait_gathers(slot)

    lane2 = lax.iota(jnp.int32, 16) * 2

    def px_body(i, carry2):
      wA = w_v[slot, 0, pl.ds(i, 16)][0]
      wB = w_v[slot, 1, pl.ds(i, 16)][0]
      wC = w_v[slot, 2, pl.ds(i, 16)][0]
      wD = w_v[slot, 3, pl.ds(i, 16)][0]
      orow = obuf.at[i]
      for c in range(C // 32):
        sA = pl.ds(c * 32, 32)
        sC = pl.ds(C + c * 32, 32)
        aE, aO = plsc.unpack(gbuf[slot, 0, i, sA],
                             format=plsc.PackFormat.INTERLEAVED)
        cE, cO = plsc.unpack(gbuf[slot, 0, i, sC],
                             format=plsc.PackFormat.INTERLEAVED)
        bE, bO = plsc.unpack(gbuf[slot, 1, i, sA],
                             format=plsc.PackFormat.INTERLEAVED)
        dE, dO = plsc.unpack(gbuf[slot, 1, i, sC],
                             format=plsc.PackFormat.INTERLEAVED)
        plsc.store_scatter(orow, [lane2 + (c * 32)],
                           aE * wA + bE * wB + cE * wC + dE * wD)
        plsc.store_scatter(orow, [lane2 + (c * 32 + 1)],
                           aO * wA + bO * wB + cO * wC + dO * wD)
      return carry2

    lax.fori_loop(0, CHUNK, px_body, 0)

    pg = p0 + k * CHUNK
    lp = pg - (wid // 4) * (H * W)
    pltpu.sync_copy(obuf, out_hbm.at[wid // 4, lp // W, pl.ds(lp % W, CHUNK)])
    return carry

  lax.fori_loop(0, CHUNKS_PER_W, chunk_body, 0)


@jax.jit
def kernel(image, affine_transforms):
  flat = image.reshape(B * H * W, C).astype(jnp.bfloat16)
  flat_pad = jnp.concatenate([flat, flat[-1:]], axis=0)
  table2 = jnp.concatenate([flat_pad[:-1], flat_pad[1:]], axis=1)

  x_lin = jnp.linspace(-1.0, 1.0, WO)
  y_lin = jnp.linspace(-1.0, 1.0, HO)
  xg, yg = jnp.meshgrid(x_lin, y_lin)
  grid = jnp.concatenate(
      [xg.reshape(-1), yg.reshape(-1), jnp.ones(HO * WO, dtype=jnp.float32)],
      axis=0)
  grids = jnp.tile(grid.reshape(1, 3, HO * WO), (B, 1, 1))
  theta = affine_transforms.reshape(B, 2, 3)
  grids = jnp.einsum('bij,bjk->bik', theta, grids)
  x = grids[:, 0:1, :].reshape(-1).astype(jnp.float32)
  y = grids[:, 1:2, :].reshape(-1).astype(jnp.float32)

  call = pl.kernel(
      _body,
      out_type=jax.ShapeDtypeStruct((B, HO, WO, C), jnp.float32),
      mesh=plsc.VectorSubcoreMesh(core_axis_name="c", subcore_axis_name="s",
                                  num_cores=NC, num_subcores=NS),
      compiler_params=pltpu.CompilerParams(use_tc_tiling_on_sc=False,
                                           needs_layout_passes=False),
      scratch_types=[
          pltpu.VMEM((PPW,), jnp.float32),
          pltpu.VMEM((PPW,), jnp.float32),
          pltpu.VMEM((2, CHUNK), jnp.int32),
          pltpu.VMEM((2, CHUNK), jnp.int32),
          pltpu.VMEM((2, 4, CHUNK + 16), jnp.float32),
          pltpu.VMEM((2, 2, CHUNK, C2), jnp.bfloat16),
          pltpu.VMEM((CHUNK, C), jnp.float32),
          pltpu.SemaphoreType.DMA((2,)),
      ],
  )
  return call(table2, x, y)
